# baseline (speedup 1.0000x reference)
import functools

import numpy as np
import jax
import jax.numpy as jnp
from jax import lax
from jax.experimental import pallas as pl
from jax.experimental.pallas import tpu as pltpu
from jax.experimental.pallas import tpu_sc as plsc

B = 16384
F = 26
V = 100000
D = 16
P = F * (F - 1) // 2

_row, _col = np.triu_indices(F, k=1)
_pid = {(i, j): p for p, (i, j) in enumerate(zip(_row, _col))}
_GAP_PERM = np.asarray(
    [_pid[(i, i + g)] for g in range(1, F) for i in range(F - g)], dtype=np.int32
)

_GATHER_WINDOW = 128
_VC = V


_VBP = 12544


def _tp_body(tt_ref, out_ref):
    pieces = []
    for sl in range(8):
        a = sl * _VBP
        w = min(V - a, _VBP)
        p = tt_ref[0, :, a:a + w]
        if w < _VBP:
            p = jnp.concatenate(
                [p, jnp.zeros((D, _VBP - w), jnp.float32)], axis=1)
        pieces.append(p)
    z = jnp.concatenate(pieces, axis=0)
    out_ref[0] = z.T


def _tc_detranspose(tt):
    return pl.pallas_call(
        _tp_body,
        grid=(F,),
        in_specs=[pl.BlockSpec((1, D, V), lambda f: (f, 0, 0))],
        out_specs=pl.BlockSpec((1, _VBP, 8 * D), lambda f: (f, 0, 0)),
        out_shape=jax.ShapeDtypeStruct((F, _VBP, 8 * D), jnp.float32),
    )(tt)


def _sc_gather(tables_flat, flat_idx):
    n = flat_idx.shape[1]
    mesh = plsc.VectorSubcoreMesh(core_axis_name="core", subcore_axis_name="subcore")

    @functools.partial(
        pl.kernel,
        out_type=jax.ShapeDtypeStruct((n, D), jnp.float32),
        mesh=mesh,
        compiler_params=pltpu.CompilerParams(use_tc_tiling_on_sc=False),
    )
    def k(x_hbm, i_hbm, o_hbm):
        def body(i_vmem, o_vmem):
            pltpu.sync_copy(x_hbm.at[i_vmem.at[0]], o_vmem)

        pltpu.emit_pipeline(
            body,
            grid=(n // _GATHER_WINDOW,),
            in_specs=[pl.BlockSpec((1, _GATHER_WINDOW), index_map=lambda i: (0, i))],
            out_specs=[pl.BlockSpec((_GATHER_WINDOW, D), index_map=lambda i: (i, 0))],
            core_axis_name=("core", "subcore"),
            dimension_semantics=(pltpu.PARALLEL,),
        )(i_hbm, o_hbm)

    return k(tables_flat, flat_idx)


def _mlp_body(xw_ref, w0a_ref, w0r_ref, w1_ref, w2_ref, b0_ref, b1_ref, b2_ref,
              out_ref):
    bf = jnp.bfloat16
    xw = xw_ref[...][:, :F * D]
    xwb = xw.astype(bf)
    zs = []
    for g in range(1, F):
        ncols = F - g
        w = D * ncols
        prod = (xw[:, :w] * xw[:, D * g:]).astype(bf)
        rgrp = lax.broadcasted_iota(jnp.int32, (w, ncols), 0) // D
        cid = lax.broadcasted_iota(jnp.int32, (w, ncols), 1)
        sel = (rgrp == cid).astype(bf)
        zs.append(jnp.dot(prod, sel, preferred_element_type=jnp.float32))
    ip = jnp.concatenate(zs, axis=1).astype(bf)
    h = jnp.dot(xwb, w0a_ref[...].astype(bf), preferred_element_type=jnp.float32)
    h = h + jnp.dot(ip, w0r_ref[...].astype(bf),
                    preferred_element_type=jnp.float32)
    h = jnp.maximum(h + b0_ref[...], 0.0).astype(bf)
    h = jnp.dot(h, w1_ref[...].astype(bf),
                preferred_element_type=jnp.float32) + b1_ref[...]
    h = jnp.maximum(h, 0.0).astype(bf)
    o = jnp.dot(h, w2_ref[...].astype(bf),
                preferred_element_type=jnp.float32) + b2_ref[...]
    out_ref[...] = jax.nn.sigmoid(o)


def _tc_mlp(xw, w0a, w0r, w1, w2, b0, b1, b2, block_b=1024):
    nb = xw.shape[0] // block_b
    full = lambda shape: pl.BlockSpec(shape, lambda i: (0, 0))
    return pl.pallas_call(
        _mlp_body,
        grid=(nb,),
        in_specs=[
            pl.BlockSpec((block_b, 512), lambda i: (i, 0)),
            full(w0a.shape),
            full(w0r.shape),
            full(w1.shape),
            full(w2.shape),
            full(b0.shape),
            full(b1.shape),
            full(b2.shape),
        ],
        out_specs=pl.BlockSpec((block_b, 1), lambda i: (i, 0)),
        out_shape=jax.ShapeDtypeStruct((xw.shape[0], 1), jnp.float32),
    )(xw, w0a, w0r, w1, w2, b0, b1, b2)


def kernel(indices, tables, w0, b0, w1, b1, w2, b2):
    idx = indices.astype(jnp.int32)
    foff = jnp.arange(F, dtype=jnp.int32) * _VBP
    flat_idx = (foff + idx % _VBP) * 8 + idx // _VBP
    flat_idx = jnp.concatenate(
        [flat_idx, jnp.zeros((B, 32 - F), jnp.int32)], axis=1)
    flat_idx = flat_idx.reshape(1, B * 32)
    tt = jnp.transpose(tables, (0, 2, 1))
    lines = _tc_detranspose(tt)
    tables_flat = lines.reshape(F * _VBP * 8, D)
    w0a = w0[: F * D]
    w0r = w0[F * D:][jnp.asarray(_GAP_PERM)]
    b0r, b1r, b2r = b0.reshape(1, -1), b1.reshape(1, -1), b2.reshape(1, -1)
    nsplit = 4
    bh = B // nsplit
    hn = bh * 32
    outs = []
    for h in range(nsplit):
        fi_h = lax.slice(flat_idx, (0, h * hn), (1, (h + 1) * hn))
        g_h = _sc_gather(tables_flat, fi_h)
        xw_h = g_h.reshape(bh, 512)
        outs.append(_tc_mlp(xw_h, w0a, w0r, w1, w2, b0r, b1r, b2r))
    return jnp.concatenate(outs, axis=0).reshape(B)


# speedup vs baseline: 2.0013x; 2.0013x over previous
import functools

import numpy as np
import jax
import jax.numpy as jnp
from jax import lax
from jax.experimental import pallas as pl
from jax.experimental.pallas import tpu as pltpu
from jax.experimental.pallas import tpu_sc as plsc

B = 16384
F = 26
V = 100000
D = 16
P = F * (F - 1) // 2

_row, _col = np.triu_indices(F, k=1)
_pid = {(i, j): p for p, (i, j) in enumerate(zip(_row, _col))}
_GAP_PERM = np.asarray(
    [_pid[(i, i + g)] for g in range(1, F) for i in range(F - g)], dtype=np.int32
)

_GATHER_WINDOW = 128
_VC = V


_VBP = 12544


def _tp_body(tt_ref, out_ref):
    pieces = []
    for sl in range(8):
        a = sl * _VBP
        w = min(V - a, _VBP)
        p = tt_ref[0, :, a:a + w]
        if w < _VBP:
            p = jnp.concatenate(
                [p, jnp.zeros((D, _VBP - w), jnp.float32)], axis=1)
        pieces.append(p)
    z = jnp.concatenate(pieces, axis=0)
    out_ref[0] = z.T


def _tc_detranspose(tt):
    return pl.pallas_call(
        _tp_body,
        grid=(F,),
        in_specs=[pl.BlockSpec((1, D, V), lambda f: (f, 0, 0))],
        out_specs=pl.BlockSpec((1, _VBP, 8 * D), lambda f: (f, 0, 0)),
        out_shape=jax.ShapeDtypeStruct((F, _VBP, 8 * D), jnp.float32),
    )(tt)


def _sc_gather(tables_flat, flat_idx):
    n = flat_idx.shape[1]
    mesh = plsc.VectorSubcoreMesh(core_axis_name="core", subcore_axis_name="subcore")

    @functools.partial(
        pl.kernel,
        out_type=jax.ShapeDtypeStruct((n, D), jnp.float32),
        mesh=mesh,
        compiler_params=pltpu.CompilerParams(use_tc_tiling_on_sc=False),
    )
    def k(x_hbm, i_hbm, o_hbm):
        def body(i_vmem, o_vmem):
            pltpu.sync_copy(x_hbm.at[i_vmem.at[0]], o_vmem)

        pltpu.emit_pipeline(
            body,
            grid=(n // _GATHER_WINDOW,),
            in_specs=[pl.BlockSpec((1, _GATHER_WINDOW), index_map=lambda i: (0, i))],
            out_specs=[pl.BlockSpec((_GATHER_WINDOW, D), index_map=lambda i: (i, 0))],
            core_axis_name=("core", "subcore"),
            dimension_semantics=(pltpu.PARALLEL,),
        )(i_hbm, o_hbm)

    return k(tables_flat, flat_idx)


def _mlp_body(xw_ref, w0a_ref, w0r_ref, w1_ref, w2_ref, b0_ref, b1_ref, b2_ref,
              out_ref):
    bf = jnp.bfloat16
    xw = xw_ref[...][:, :F * D]
    xwb = xw.astype(bf)
    zs = []
    for g in range(1, F):
        ncols = F - g
        w = D * ncols
        prod = (xw[:, :w] * xw[:, D * g:]).astype(bf)
        rgrp = lax.broadcasted_iota(jnp.int32, (w, ncols), 0) // D
        cid = lax.broadcasted_iota(jnp.int32, (w, ncols), 1)
        sel = (rgrp == cid).astype(bf)
        zs.append(jnp.dot(prod, sel, preferred_element_type=jnp.float32))
    ip = jnp.concatenate(zs, axis=1).astype(bf)
    h = jnp.dot(xwb, w0a_ref[...].astype(bf), preferred_element_type=jnp.float32)
    h = h + jnp.dot(ip, w0r_ref[...].astype(bf),
                    preferred_element_type=jnp.float32)
    h = jnp.maximum(h + b0_ref[...], 0.0).astype(bf)
    h = jnp.dot(h, w1_ref[...].astype(bf),
                preferred_element_type=jnp.float32) + b1_ref[...]
    h = jnp.maximum(h, 0.0).astype(bf)
    o = jnp.dot(h, w2_ref[...].astype(bf),
                preferred_element_type=jnp.float32) + b2_ref[...]
    out_ref[...] = jax.nn.sigmoid(o)


def _tc_mlp(xw, w0a, w0r, w1, w2, b0, b1, b2, block_b=1024):
    nb = xw.shape[0] // block_b
    full = lambda shape: pl.BlockSpec(shape, lambda i: (0, 0))
    return pl.pallas_call(
        _mlp_body,
        grid=(nb,),
        in_specs=[
            pl.BlockSpec((block_b, 512), lambda i: (i, 0)),
            full(w0a.shape),
            full(w0r.shape),
            full(w1.shape),
            full(w2.shape),
            full(b0.shape),
            full(b1.shape),
            full(b2.shape),
        ],
        out_specs=pl.BlockSpec((block_b, 1), lambda i: (i, 0)),
        out_shape=jax.ShapeDtypeStruct((xw.shape[0], 1), jnp.float32),
    )(xw, w0a, w0r, w1, w2, b0, b1, b2)


def kernel(indices, tables, w0, b0, w1, b1, w2, b2):
    idx = indices.astype(jnp.int32)
    foff = jnp.arange(F, dtype=jnp.int32) * _VBP
    flat_idx = (foff + idx % _VBP) * 8 + idx // _VBP
    flat_idx = jnp.concatenate(
        [flat_idx, jnp.broadcast_to(flat_idx[:, :1], (B, 32 - F))], axis=1)
    flat_idx = flat_idx.reshape(1, B * 32)
    tt = jnp.transpose(tables, (0, 2, 1))
    lines = _tc_detranspose(tt)
    tables_flat = lines.reshape(F * _VBP * 8, D)
    w0a = w0[: F * D]
    w0r = w0[F * D:][jnp.asarray(_GAP_PERM)]
    b0r, b1r, b2r = b0.reshape(1, -1), b1.reshape(1, -1), b2.reshape(1, -1)
    nsplit = 4
    bh = B // nsplit
    hn = bh * 32
    outs = []
    for h in range(nsplit):
        fi_h = lax.slice(flat_idx, (0, h * hn), (1, (h + 1) * hn))
        g_h = _sc_gather(tables_flat, fi_h)
        xw_h = g_h.reshape(bh, 512)
        outs.append(_tc_mlp(xw_h, w0a, w0r, w1, w2, b0r, b1r, b2r))
    return jnp.concatenate(outs, axis=0).reshape(B)
